# baseline (device time: 9906 ns/iter reference)
import jax
import jax.numpy as jnp
from jax import lax
from jax.experimental import pallas as pl
from jax.experimental.pallas import tpu as pltpu

N_DEV = 4
EPS = 1e-5
C = 2


def kernel(x, gamma, beta):
    m, n_loc = x.shape
    n_glob = n_loc * N_DEV
    mc = m // C

    gamma2 = gamma.reshape(1, n_loc)
    beta2 = beta.reshape(1, n_loc)

    def body(x_hbm, g_hbm, b_hbm, o_hbm,
             xv, gv, bv, ov, stats_ref, send_sems, recv_sems, lsems):
        me = lax.axis_index("i")

        cp_x = []
        for c in range(C):
            cp = pltpu.make_async_copy(
                x_hbm.at[pl.ds(c * mc, mc), :],
                xv.at[pl.ds(c * mc, mc), :],
                lsems.at[c],
            )
            cp.start()
            cp_x.append(cp)
        cp_g = pltpu.make_async_copy(g_hbm, gv, lsems.at[C])
        cp_g.start()
        cp_b = pltpu.make_async_copy(b_hbm, bv, lsems.at[C + 1])
        cp_b.start()

        barrier_sem = pltpu.get_barrier_semaphore()
        for k in range(1, N_DEV):
            peer = lax.rem(me + k, N_DEV)
            pl.semaphore_signal(
                barrier_sem, inc=1,
                device_id=(peer,), device_id_type=pl.DeviceIdType.MESH,
            )

        sends = []
        for c in range(C):
            cp_x[c].wait()
            xfc = xv[c * mc:(c + 1) * mc, :]
            s1 = jnp.sum(xfc, axis=1)
            s2 = jnp.sum(xfc * xfc, axis=1)
            my_stats = jnp.stack([s1, s2], axis=0)
            stats_ref.at[c][pl.ds(me, 1)] = my_stats[None, :, :]

            if c == 0:
                pl.semaphore_wait(barrier_sem, N_DEV - 1)

            for k in range(1, N_DEV):
                peer = lax.rem(me + k, N_DEV)
                rdma = pltpu.make_async_remote_copy(
                    src_ref=stats_ref.at[c, me],
                    dst_ref=stats_ref.at[c, me],
                    send_sem=send_sems.at[c, k - 1],
                    recv_sem=recv_sems.at[c, me],
                    device_id=(peer,),
                    device_id_type=pl.DeviceIdType.MESH,
                )
                rdma.start()
                sends.append(rdma)

        cp_g.wait()
        cp_b.wait()
        g32 = gv[:, :]
        b32 = bv[:, :]

        out_cps = []
        for c in range(C):
            for k in range(1, N_DEV):
                peer = lax.rem(me + k, N_DEV)
                recv = pltpu.make_async_remote_copy(
                    src_ref=stats_ref.at[c, peer],
                    dst_ref=stats_ref.at[c, peer],
                    send_sem=send_sems.at[c, k - 1],
                    recv_sem=recv_sems.at[c, peer],
                    device_id=(peer,),
                    device_id_type=pl.DeviceIdType.MESH,
                )
                recv.wait_recv()

            total = jnp.sum(stats_ref[c], axis=0)
            mean = total[0] * (1.0 / n_glob)
            var = total[1] * (1.0 / n_glob) - mean * mean
            inv = lax.rsqrt(var + EPS)
            xfc = xv[c * mc:(c + 1) * mc, :]
            xn = (xfc - mean[:, None]) * inv[:, None]
            ov[c * mc:(c + 1) * mc, :] = (xn * g32 + b32).astype(jnp.bfloat16)

            cp_o = pltpu.make_async_copy(
                ov.at[pl.ds(c * mc, mc), :],
                o_hbm.at[pl.ds(c * mc, mc), :],
                lsems.at[C + 2 + c],
            )
            cp_o.start()
            out_cps.append(cp_o)

        for cp in out_cps:
            cp.wait()
        for rdma in sends:
            rdma.wait_send()

    return pl.pallas_call(
        body,
        out_shape=jax.ShapeDtypeStruct((m, n_loc), jnp.bfloat16),
        in_specs=[pl.BlockSpec(memory_space=pl.ANY)] * 3,
        out_specs=pl.BlockSpec(memory_space=pl.ANY),
        scratch_shapes=[
            pltpu.VMEM((m, n_loc), jnp.float32),
            pltpu.VMEM((1, n_loc), jnp.float32),
            pltpu.VMEM((1, n_loc), jnp.float32),
            pltpu.VMEM((m, n_loc), jnp.bfloat16),
            pltpu.VMEM((C, N_DEV, 2, mc), jnp.float32),
            pltpu.SemaphoreType.DMA((C, N_DEV - 1)),
            pltpu.SemaphoreType.DMA((C, N_DEV)),
            pltpu.SemaphoreType.DMA((2 * C + 2,)),
        ],
        compiler_params=pltpu.CompilerParams(collective_id=0),
    )(x, gamma2, beta2)


# device time: 9876 ns/iter; 1.0030x vs baseline; 1.0030x over previous
import jax
import jax.numpy as jnp
from jax import lax
from jax.experimental import pallas as pl
from jax.experimental.pallas import tpu as pltpu

N_DEV = 4
EPS = 1e-5
C = 2


def kernel(x, gamma, beta):
    m, n_loc = x.shape
    n_glob = n_loc * N_DEV
    mc = m // C

    gamma2 = gamma.reshape(1, n_loc)
    beta2 = beta.reshape(1, n_loc)

    def body(x_hbm, g_hbm, b_hbm, o_hbm,
             xv, gv, bv, ov, stats_ref, send_sems, recv_sems, lsems):
        me = lax.axis_index("i")

        cp_x = []
        for c in range(C):
            cp = pltpu.make_async_copy(
                x_hbm.at[pl.ds(c * mc, mc), :],
                xv.at[pl.ds(c * mc, mc), :],
                lsems.at[c],
            )
            cp.start()
            cp_x.append(cp)
        cp_g = pltpu.make_async_copy(g_hbm, gv, lsems.at[C])
        cp_g.start()
        cp_b = pltpu.make_async_copy(b_hbm, bv, lsems.at[C + 1])
        cp_b.start()

        barrier_sem = pltpu.get_barrier_semaphore()
        for k in range(1, N_DEV):
            peer = lax.rem(me + k, N_DEV)
            pl.semaphore_signal(
                barrier_sem, inc=1,
                device_id=(peer,), device_id_type=pl.DeviceIdType.MESH,
            )

        sends = []
        for c in range(C):
            cp_x[c].wait()
            xfc = xv[c * mc:(c + 1) * mc, :]
            s1 = jnp.sum(xfc, axis=1)
            s2 = jnp.sum(xfc * xfc, axis=1)
            my_stats = jnp.stack([s1, s2], axis=0)
            stats_ref.at[c][pl.ds(me, 1)] = my_stats[None, :, :]

            if c == 0:
                pl.semaphore_wait(barrier_sem, N_DEV - 1)

            for k in range(1, N_DEV):
                peer = lax.rem(me + k, N_DEV)
                rdma = pltpu.make_async_remote_copy(
                    src_ref=stats_ref.at[c, me],
                    dst_ref=stats_ref.at[c, me],
                    send_sem=send_sems.at[c, k - 1],
                    recv_sem=recv_sems.at[c, me],
                    device_id=(peer,),
                    device_id_type=pl.DeviceIdType.MESH,
                )
                rdma.start()
                sends.append(rdma)

        cp_g.wait()
        cp_b.wait()
        g32 = gv[:, :]
        b32 = bv[:, :]

        out_cps = []
        for c in range(C):
            for k in range(1, N_DEV):
                peer = lax.rem(me + k, N_DEV)
                recv = pltpu.make_async_remote_copy(
                    src_ref=stats_ref.at[c, peer],
                    dst_ref=stats_ref.at[c, peer],
                    send_sem=send_sems.at[c, k - 1],
                    recv_sem=recv_sems.at[c, peer],
                    device_id=(peer,),
                    device_id_type=pl.DeviceIdType.MESH,
                )
                recv.wait_recv()

            total = jnp.sum(stats_ref[c], axis=0)
            mean = total[0] * (1.0 / n_glob)
            var = total[1] * (1.0 / n_glob) - mean * mean
            inv = lax.rsqrt(var + EPS)
            xfc = xv[c * mc:(c + 1) * mc, :]
            xn = (xfc - mean[:, None]) * inv[:, None]
            ov[c * mc:(c + 1) * mc, :] = (xn * g32 + b32).astype(jnp.bfloat16)

            cp_o = pltpu.make_async_copy(
                ov.at[pl.ds(c * mc, mc), :],
                o_hbm.at[pl.ds(c * mc, mc), :],
                lsems.at[C + 2 + c],
            )
            cp_o.start()
            out_cps.append(cp_o)

        for cp in out_cps:
            cp.wait()
        for rdma in sends:
            rdma.wait_send()

    return pl.pallas_call(
        body,
        out_shape=jax.ShapeDtypeStruct((m, n_loc), jnp.bfloat16),
        in_specs=[pl.BlockSpec(memory_space=pltpu.MemorySpace.HBM)] * 3,
        out_specs=pl.BlockSpec(memory_space=pltpu.MemorySpace.HBM),
        scratch_shapes=[
            pltpu.VMEM((m, n_loc), jnp.float32),
            pltpu.VMEM((1, n_loc), jnp.float32),
            pltpu.VMEM((1, n_loc), jnp.float32),
            pltpu.VMEM((m, n_loc), jnp.bfloat16),
            pltpu.VMEM((C, N_DEV, 2, mc), jnp.float32),
            pltpu.SemaphoreType.DMA((C, N_DEV - 1)),
            pltpu.SemaphoreType.DMA((C, N_DEV)),
            pltpu.SemaphoreType.DMA((2 * C + 2,)),
        ],
        compiler_params=pltpu.CompilerParams(collective_id=0),
    )(x, gamma2, beta2)


# device time: 8960 ns/iter; 1.1056x vs baseline; 1.1022x over previous
import jax
import jax.numpy as jnp
from jax import lax
from jax.experimental import pallas as pl
from jax.experimental.pallas import tpu as pltpu

N_DEV = 4
EPS = 1e-5
C = 2


def kernel(x, gamma, beta):
    m, n_loc = x.shape
    n_glob = n_loc * N_DEV
    mc = m // C

    gb = jnp.stack([gamma, beta], axis=0)

    def body(x_ref, gb_ref, o_ref, stats_ref, send_sems, recv_sems):
        me = lax.axis_index("i")

        barrier_sem = pltpu.get_barrier_semaphore()
        for k in range(1, N_DEV):
            peer = lax.rem(me + k, N_DEV)
            pl.semaphore_signal(
                barrier_sem, inc=1,
                device_id=(peer,), device_id_type=pl.DeviceIdType.MESH,
            )

        sends = []
        for c in range(C):
            xfc = x_ref[c * mc:(c + 1) * mc, :]
            s1 = jnp.sum(xfc, axis=1)
            s2 = jnp.sum(xfc * xfc, axis=1)
            my_stats = jnp.stack([s1, s2], axis=0)
            stats_ref.at[c][pl.ds(me, 1)] = my_stats[None, :, :]

            if c == 0:
                pl.semaphore_wait(barrier_sem, N_DEV - 1)

            for k in range(1, N_DEV):
                peer = lax.rem(me + k, N_DEV)
                rdma = pltpu.make_async_remote_copy(
                    src_ref=stats_ref.at[c, me],
                    dst_ref=stats_ref.at[c, me],
                    send_sem=send_sems.at[c, k - 1],
                    recv_sem=recv_sems.at[c, me],
                    device_id=(peer,),
                    device_id_type=pl.DeviceIdType.MESH,
                )
                rdma.start()
                sends.append(rdma)

        g32 = gb_ref[0:1, :]
        b32 = gb_ref[1:2, :]

        for c in range(C):
            for k in range(1, N_DEV):
                peer = lax.rem(me + k, N_DEV)
                recv = pltpu.make_async_remote_copy(
                    src_ref=stats_ref.at[c, peer],
                    dst_ref=stats_ref.at[c, peer],
                    send_sem=send_sems.at[c, k - 1],
                    recv_sem=recv_sems.at[c, peer],
                    device_id=(peer,),
                    device_id_type=pl.DeviceIdType.MESH,
                )
                recv.wait_recv()

            total = jnp.sum(stats_ref[c], axis=0)
            mean = total[0] * (1.0 / n_glob)
            var = total[1] * (1.0 / n_glob) - mean * mean
            inv = lax.rsqrt(var + EPS)
            xfc = x_ref[c * mc:(c + 1) * mc, :]
            xn = (xfc - mean[:, None]) * inv[:, None]
            o_ref[c * mc:(c + 1) * mc, :] = (xn * g32 + b32).astype(jnp.bfloat16)

        for rdma in sends:
            rdma.wait_send()

    return pl.pallas_call(
        body,
        out_shape=jax.ShapeDtypeStruct((m, n_loc), jnp.bfloat16),
        in_specs=[pl.BlockSpec(memory_space=pltpu.VMEM)] * 2,
        out_specs=pl.BlockSpec(memory_space=pltpu.VMEM),
        scratch_shapes=[
            pltpu.VMEM((C, N_DEV, 2, mc), jnp.float32),
            pltpu.SemaphoreType.DMA((C, N_DEV - 1)),
            pltpu.SemaphoreType.DMA((C, N_DEV)),
        ],
        compiler_params=pltpu.CompilerParams(collective_id=0),
    )(x, gb)


# device time: 8837 ns/iter; 1.1210x vs baseline; 1.0139x over previous
import jax
import jax.numpy as jnp
from jax import lax
from jax.experimental import pallas as pl
from jax.experimental.pallas import tpu as pltpu

N_DEV = 4
EPS = 1e-5
C = 2


def kernel(x, gamma, beta):
    m, n_loc = x.shape
    n_glob = n_loc * N_DEV
    mc = m // C

    gb = jnp.stack([gamma, beta], axis=0)

    def body(x_ref, gb_ref, o_ref, stats_ref, send_sems, recv_sems):
        me = lax.axis_index("i")

        barrier_sem = pltpu.get_barrier_semaphore()
        for k in range(1, N_DEV):
            peer = lax.rem(me + k, N_DEV)
            pl.semaphore_signal(
                barrier_sem, inc=1,
                device_id=(peer,), device_id_type=pl.DeviceIdType.MESH,
            )

        sends = []
        for c in range(C):
            xfc = x_ref[c * mc:(c + 1) * mc, :].astype(jnp.float32)
            s1 = jnp.sum(xfc, axis=1)
            s2 = jnp.sum(xfc * xfc, axis=1)
            my_stats = jnp.stack([s1, s2], axis=0)
            stats_ref.at[c][pl.ds(me, 1)] = my_stats[None, :, :]

            if c == 0:
                pl.semaphore_wait(barrier_sem, N_DEV - 1)

            for k in range(1, N_DEV):
                peer = lax.rem(me + k, N_DEV)
                rdma = pltpu.make_async_remote_copy(
                    src_ref=stats_ref.at[c, me],
                    dst_ref=stats_ref.at[c, me],
                    send_sem=send_sems.at[c, k - 1],
                    recv_sem=recv_sems.at[c, me],
                    device_id=(peer,),
                    device_id_type=pl.DeviceIdType.MESH,
                )
                rdma.start()
                sends.append(rdma)

        g32 = gb_ref[0:1, :]
        b32 = gb_ref[1:2, :]

        for c in range(C):
            for k in range(1, N_DEV):
                peer = lax.rem(me + k, N_DEV)
                recv = pltpu.make_async_remote_copy(
                    src_ref=stats_ref.at[c, peer],
                    dst_ref=stats_ref.at[c, peer],
                    send_sem=send_sems.at[c, k - 1],
                    recv_sem=recv_sems.at[c, peer],
                    device_id=(peer,),
                    device_id_type=pl.DeviceIdType.MESH,
                )
                recv.wait_recv()

            total = jnp.sum(stats_ref[c], axis=0)
            mean = total[0] * (1.0 / n_glob)
            var = total[1] * (1.0 / n_glob) - mean * mean
            inv = lax.rsqrt(var + EPS)
            xfc = x_ref[c * mc:(c + 1) * mc, :].astype(jnp.float32)
            xn = (xfc - mean[:, None]) * inv[:, None]
            o_ref[c * mc:(c + 1) * mc, :] = (xn * g32 + b32).astype(jnp.bfloat16)

        for rdma in sends:
            rdma.wait_send()

    return pl.pallas_call(
        body,
        out_shape=jax.ShapeDtypeStruct((m, n_loc), jnp.bfloat16),
        in_specs=[pl.BlockSpec(memory_space=pltpu.VMEM)] * 2,
        out_specs=pl.BlockSpec(memory_space=pltpu.VMEM),
        scratch_shapes=[
            pltpu.VMEM((C, N_DEV, 2, mc), jnp.float32),
            pltpu.SemaphoreType.DMA((C, N_DEV - 1)),
            pltpu.SemaphoreType.DMA((C, N_DEV)),
        ],
        compiler_params=pltpu.CompilerParams(collective_id=0),
    )(x, gb)
